# Initial kernel scaffold; baseline (speedup 1.0000x reference)
import jax
import jax.numpy as jnp
from jax.experimental import pallas as pl


def kernel(x_article, x_tweet, x_user, batch_article, batch_tweet, batch_user, W, b):
    raise NotImplementedError("write your pallas kernel here")


# speedup vs baseline: 4.0921x; 4.0921x over previous
import functools

import jax
import jax.numpy as jnp
from jax import lax
from jax.experimental import pallas as pl
from jax.experimental.pallas import tpu as pltpu
from jax.experimental.pallas import tpu_sc as plsc

N = 100000
D = 128
S = 512
OUT = 128

NC = 2
NS = 16
NW = NC * NS
LANES = 16

CHUNK = 80
NCHUNKS = N // CHUNK


def _sc_pool(x_a, x_t, x_u, b_a, b_t, b_u):
    mesh = plsc.VectorSubcoreMesh(core_axis_name="c", subcore_axis_name="s")

    out_type = (
        [jax.ShapeDtypeStruct((NC, S, D), jnp.float32) for _ in range(3)]
        + [jax.ShapeDtypeStruct((NC, S, D), jnp.float32) for _ in range(3)]
    )

    scratch = dict(
        acc0=pltpu.VMEM_SHARED((S, D), jnp.float32),
        acc1=pltpu.VMEM_SHARED((S, D), jnp.float32),
        acc2=pltpu.VMEM_SHARED((S, D), jnp.float32),
        cnt0=pltpu.VMEM_SHARED((S, D), jnp.float32),
        cnt1=pltpu.VMEM_SHARED((S, D), jnp.float32),
        cnt2=pltpu.VMEM_SHARED((S, D), jnp.float32),
        buf=pltpu.VMEM((CHUNK, D), jnp.float32),
        idx=pltpu.VMEM((CHUNK,), jnp.int32),
        ones=pltpu.VMEM((CHUNK, D), jnp.float32),
        zrow=pltpu.VMEM((S // NS, D), jnp.float32),
    )

    @functools.partial(pl.kernel, out_type=out_type, mesh=mesh,
                       scratch_types=scratch)
    def k(xa, xt, xu, ba, bt, bu, sa, st, su, ca, ct, cu,
          acc0, acc1, acc2, cnt0, cnt1, cnt2, buf, idx, ones, zrow):
        cid = lax.axis_index("c")
        sid = lax.axis_index("s")
        wid = sid * NC + cid
        rows_per = S // NS

        def _zero_row(r, _):
            for kk in range(D // LANES):
                zrow[r, pl.ds(kk * LANES, LANES)] = jnp.zeros((LANES,), jnp.float32)
            return _
        lax.fori_loop(0, rows_per, _zero_row, 0)

        def _one_row(r, _):
            for kk in range(D // LANES):
                ones[r, pl.ds(kk * LANES, LANES)] = jnp.ones((LANES,), jnp.float32)
            return _
        lax.fori_loop(0, CHUNK, _one_row, 0)

        row0 = sid * rows_per
        for acc in (acc0, acc1, acc2, cnt0, cnt1, cnt2):
            pltpu.sync_copy(zrow, acc.at[pl.ds(row0, rows_per)])
        plsc.subcore_barrier()

        def _process(x_hbm, b_hbm, acc, cnt):
            nchunks_here = (NCHUNKS - wid + NW - 1) // NW

            def _body(i, _):
                c = wid + i * NW
                base = c * CHUNK
                pltpu.sync_copy(b_hbm.at[pl.ds(base, CHUNK)], idx)
                pltpu.sync_copy(x_hbm.at[pl.ds(base, CHUNK)], buf)
                pltpu.sync_copy(buf, acc.at[idx], add=True)
                pltpu.sync_copy(ones, cnt.at[idx], add=True)
                return _
            lax.fori_loop(0, nchunks_here, _body, 0)

        _process(xa, ba, acc0, cnt0)
        _process(xt, bt, acc1, cnt1)
        _process(xu, bu, acc2, cnt2)
        plsc.subcore_barrier()

        for acc, out in ((acc0, sa), (acc1, st), (acc2, su),
                         (cnt0, ca), (cnt1, ct), (cnt2, cu)):
            pltpu.sync_copy(acc.at[pl.ds(row0, rows_per)],
                            out.at[cid, pl.ds(row0, rows_per)])

    return k(x_a, x_t, x_u, b_a, b_t, b_u)


def _tc_finish_body(sa, st, su, ca, ct, cu, w_ref, b_ref, out_ref):
    w = w_ref[...]
    out = jnp.broadcast_to(b_ref[...], (S, OUT))
    for a, (s_ref, c_ref) in enumerate(((sa, ca), (st, ct), (su, cu))):
        tot = s_ref[0] + s_ref[1]
        cnt = c_ref[0, :, 0:1] + c_ref[1, :, 0:1]
        mean = tot / jnp.maximum(cnt, 1.0)
        out = out + lax.dot_general(
            mean, w[:, a * D:(a + 1) * D],
            dimension_numbers=(((1,), (1,)), ((), ())),
            preferred_element_type=jnp.float32,
            precision=lax.Precision.HIGHEST,
        )
    out_ref[...] = out


def kernel(x_article, x_tweet, x_user, batch_article, batch_tweet, batch_user, W, b):
    sa, st, su, ca, ct, cu = _sc_pool(
        x_article, x_tweet, x_user, batch_article, batch_tweet, batch_user)
    return pl.pallas_call(
        _tc_finish_body,
        out_shape=jax.ShapeDtypeStruct((S, OUT), jnp.float32),
    )(sa, st, su, ca, ct, cu, W, b.reshape(1, OUT))


# speedup vs baseline: 7.4601x; 1.8230x over previous
import functools

import jax
import jax.numpy as jnp
from jax import lax
from jax.experimental import pallas as pl
from jax.experimental.pallas import tpu as pltpu
from jax.experimental.pallas import tpu_sc as plsc

N = 100000
D = 128
S = 512
OUT = 128

NC = 2
NS = 16
NW = NC * NS
LANES = 16

CHUNK = 80
NCHUNKS = N // CHUNK


def _sc_pool(x_a, x_t, x_u, b_a, b_t, b_u):
    mesh = plsc.VectorSubcoreMesh(core_axis_name="c", subcore_axis_name="s")

    out_type = (
        [jax.ShapeDtypeStruct((NC, S, D), jnp.float32) for _ in range(3)]
        + [jax.ShapeDtypeStruct((NC, S, D), jnp.float32) for _ in range(3)]
    )

    nslots = 3
    scratch = dict(
        acc0=pltpu.VMEM_SHARED((S, D), jnp.float32),
        acc1=pltpu.VMEM_SHARED((S, D), jnp.float32),
        acc2=pltpu.VMEM_SHARED((S, D), jnp.float32),
        cnt0=pltpu.VMEM_SHARED((S, D), jnp.float32),
        cnt1=pltpu.VMEM_SHARED((S, D), jnp.float32),
        cnt2=pltpu.VMEM_SHARED((S, D), jnp.float32),
        bufs=[pltpu.VMEM((CHUNK, D), jnp.float32) for _ in range(nslots)],
        idxs=[pltpu.VMEM((CHUNK,), jnp.int32) for _ in range(nslots)],
        ones=pltpu.VMEM((CHUNK, D), jnp.float32),
        zrow=pltpu.VMEM((S // NS, D), jnp.float32),
        ld_i=[pltpu.SemaphoreType.DMA for _ in range(nslots)],
        ld_b=[pltpu.SemaphoreType.DMA for _ in range(nslots)],
        sc_d=[pltpu.SemaphoreType.DMA for _ in range(nslots)],
        sc_o=[pltpu.SemaphoreType.DMA for _ in range(nslots)],
    )

    KMAIN = 39
    assert KMAIN % nslots == 0 and KMAIN * NW < NCHUNKS <= (KMAIN + 1) * NW

    @functools.partial(pl.kernel, out_type=out_type, mesh=mesh,
                       scratch_types=scratch)
    def k(xa, xt, xu, ba, bt, bu, sa, st, su, ca, ct, cu,
          acc0, acc1, acc2, cnt0, cnt1, cnt2, bufs, idxs, ones, zrow,
          ld_i, ld_b, sc_d, sc_o):
        cid = lax.axis_index("c")
        sid = lax.axis_index("s")
        wid = sid * NC + cid
        rows_per = S // NS

        def _zero_row(r, _):
            for kk in range(D // LANES):
                zrow[r, pl.ds(kk * LANES, LANES)] = jnp.zeros((LANES,), jnp.float32)
            return _
        lax.fori_loop(0, rows_per, _zero_row, 0)

        def _one_row(r, _):
            for kk in range(D // LANES):
                ones[r, pl.ds(kk * LANES, LANES)] = jnp.ones((LANES,), jnp.float32)
            return _
        lax.fori_loop(0, CHUNK, _one_row, 0)

        row0 = sid * rows_per
        for acc in (acc0, acc1, acc2, cnt0, cnt1, cnt2):
            pltpu.sync_copy(zrow, acc.at[pl.ds(row0, rows_per)])
        plsc.subcore_barrier()

        def _start_load(x_hbm, b_hbm, kc, j):
            base = (wid + kc * NW) * CHUNK
            pltpu.async_copy(b_hbm.at[pl.ds(base, CHUNK)], idxs[j], ld_i[j])
            pltpu.async_copy(x_hbm.at[pl.ds(base, CHUNK)], bufs[j], ld_b[j])

        def _wait_load(x_hbm, b_hbm, j):
            pltpu.make_async_copy(b_hbm.at[pl.ds(0, CHUNK)], idxs[j], ld_i[j]).wait()
            pltpu.make_async_copy(x_hbm.at[pl.ds(0, CHUNK)], bufs[j], ld_b[j]).wait()

        def _process(x_hbm, b_hbm, acc, cnt, phase):
            for j in range(nslots):
                _start_load(x_hbm, b_hbm, j, j)

            def _body(ip, _):
                for j in range(nslots):
                    kc = ip * nslots + j
                    _wait_load(x_hbm, b_hbm, j)
                    pltpu.async_copy(bufs[j], acc.at[idxs[j]], sc_d[j], add=True)
                    pltpu.async_copy(ones, cnt.at[idxs[j]], sc_o[j], add=True)
                    pltpu.make_async_copy(bufs[j], acc.at[idxs[j]], sc_d[j]).wait()
                    pltpu.make_async_copy(ones, cnt.at[idxs[j]], sc_o[j]).wait()
                    knext = jnp.minimum(kc + nslots, KMAIN - 1)
                    _start_load(x_hbm, b_hbm, knext, j)
                return _
            lax.fori_loop(0, KMAIN // nslots, _body, 0)

            for j in range(nslots):
                _wait_load(x_hbm, b_hbm, j)

            ntail = NCHUNKS - KMAIN * NW
            tail_rank = wid - phase * ntail
            @pl.when(jnp.logical_and(tail_rank >= 0, tail_rank < ntail))
            def _():
                base = (KMAIN * NW + tail_rank) * CHUNK
                pltpu.sync_copy(b_hbm.at[pl.ds(base, CHUNK)], idxs[0])
                pltpu.sync_copy(x_hbm.at[pl.ds(base, CHUNK)], bufs[0])
                pltpu.sync_copy(bufs[0], acc.at[idxs[0]], add=True)
                pltpu.sync_copy(ones, cnt.at[idxs[0]], add=True)

        _process(xa, ba, acc0, cnt0, 0)
        _process(xt, bt, acc1, cnt1, 1)
        _process(xu, bu, acc2, cnt2, 2)
        plsc.subcore_barrier()

        for acc, out in ((acc0, sa), (acc1, st), (acc2, su),
                         (cnt0, ca), (cnt1, ct), (cnt2, cu)):
            pltpu.sync_copy(acc.at[pl.ds(row0, rows_per)],
                            out.at[cid, pl.ds(row0, rows_per)])

    return k(x_a, x_t, x_u, b_a, b_t, b_u)


def _tc_finish_body(sa, st, su, ca, ct, cu, w_ref, b_ref, out_ref):
    w = w_ref[...]
    out = jnp.broadcast_to(b_ref[...], (S, OUT))
    for a, (s_ref, c_ref) in enumerate(((sa, ca), (st, ct), (su, cu))):
        tot = s_ref[0] + s_ref[1]
        cnt = c_ref[0, :, 0:1] + c_ref[1, :, 0:1]
        mean = tot / jnp.maximum(cnt, 1.0)
        out = out + lax.dot_general(
            mean, w[:, a * D:(a + 1) * D],
            dimension_numbers=(((1,), (1,)), ((), ())),
            preferred_element_type=jnp.float32,
            precision=lax.Precision.HIGHEST,
        )
    out_ref[...] = out


def kernel(x_article, x_tweet, x_user, batch_article, batch_tweet, batch_user, W, b):
    sa, st, su, ca, ct, cu = _sc_pool(
        x_article, x_tweet, x_user, batch_article, batch_tweet, batch_user)
    return pl.pallas_call(
        _tc_finish_body,
        out_shape=jax.ShapeDtypeStruct((S, OUT), jnp.float32),
    )(sa, st, su, ca, ct, cu, W, b.reshape(1, OUT))
